# Initial kernel scaffold; baseline (speedup 1.0000x reference)
import functools

import jax
import jax.numpy as jnp
from jax import lax
from jax.experimental import pallas as pl
from jax.experimental.pallas import tpu as pltpu
from jax.experimental.pallas import tpu_sc as plsc

S = 2048
B = 4
D = 128
L = 2048
E = 16384
NE = B * E

NC = 2
NS = 16
NW = NC * NS
EPT = NE // NW
CHUNKS = EPT // 128

LBLK = 512

_EPS = 1e-10


def _tc_body(m_ref, w_ref, a_ref):
    mb = m_ref[:, 0, :]
    wb = w_ref[...]
    scale = lax.dot_general(wb, mb, (((1,), (1,)), ((), ())),
                            preferred_element_type=jnp.float32)
    mx = jnp.max(scale, axis=1, keepdims=True)
    e = jnp.exp(scale - mx)
    z = jnp.sum(e, axis=1, keepdims=True)
    a_ref[0] = e / z


def _alpha_dense(M, W):
    return pl.pallas_call(
        _tc_body,
        grid=(B, L // LBLK),
        in_specs=[
            pl.BlockSpec((S, 1, D), lambda b, l: (0, b, 0)),
            pl.BlockSpec((LBLK, D), lambda b, l: (l, 0)),
        ],
        out_specs=pl.BlockSpec((1, LBLK, S), lambda b, l: (b, l, 0)),
        out_shape=jax.ShapeDtypeStruct((B, L, S), jnp.float32),
    )(M, W)


def _sc_body(a_hbm, fi_hbm, tg_hbm, out_hbm, tag_hbm,
             fi_v, tg_v, al_v, wn_v, ma_v, rl_v, sc_v, dv_v, zb_v,
             d_sh, sem, zsem):
    c = lax.axis_index("c")
    s = lax.axis_index("s")

    def _z(i, carry):
        zb_v[pl.ds(i * 16, 16)] = jnp.zeros((16,), jnp.float32)
        return carry
    lax.fori_loop(0, zb_v.shape[0] // 16, _z, 0)

    @pl.when(s == 0)
    def _():
        pltpu.sync_copy(zb_v.at[pl.ds(0, B * L // NC)], d_sh)

    pltpu.sync_copy(fi_hbm.at[c, s], fi_v)
    pltpu.sync_copy(tg_hbm.at[c, s], tg_v)

    gathers = [pltpu.async_copy(a_hbm.at[fi_v.at[k]], al_v.at[k], sem)
               for k in range(CHUNKS)]
    for g in gathers:
        g.wait()

    scats = [pltpu.async_copy(tg_v.at[k], tag_hbm.at[fi_v.at[k]], sem)
             for k in range(CHUNKS)]
    for g in scats:
        g.wait()
    plsc.subcore_barrier()

    wins = [pltpu.async_copy(tag_hbm.at[fi_v.at[k]], wn_v.at[k], sem)
            for k in range(CHUNKS)]
    for g in wins:
        g.wait()

    row_base = c * (B * L // NC)
    for k in range(CHUNKS):
        for j in range(8):
            sl = pl.ds(j * 16, 16)
            fi16 = fi_v[k, sl]
            canon = wn_v[k, sl] == tg_v[k, sl]
            ma_v[k, sl] = jnp.where(canon, al_v[k, sl], 0.0)
            rl_v[k, sl] = lax.shift_right_logical(fi16, 11) - row_base

    adds = [pltpu.async_copy(ma_v.at[k], d_sh.at[rl_v.at[k]], sem, add=True)
            for k in range(CHUNKS)]
    for g in adds:
        g.wait()
    plsc.subcore_barrier()

    pltpu.sync_copy(d_sh, dv_v)

    for k in range(CHUNKS):
        for j in range(8):
            sl = pl.ds(j * 16, 16)
            dval = plsc.load_gather(dv_v, [rl_v[k, sl]])
            denom = dval * (1.0 - _EPS) + _EPS
            sc_v[k, sl] = al_v[k, sl] / denom

    zn = zb_v.shape[0]
    base = c * (B * L * S // NC) + s * (B * L * S // NW)
    nz = (B * L * S // NW) // zn
    zcopies = [pltpu.async_copy(zb_v, out_hbm.at[pl.ds(base + q * zn, zn)], zsem)
               for q in range(nz)]
    for g in zcopies:
        g.wait()
    plsc.subcore_barrier()

    outs = [pltpu.async_copy(sc_v.at[k], out_hbm.at[fi_v.at[k]], sem)
            for k in range(CHUNKS)]
    for g in outs:
        g.wait()


@functools.partial(
    pl.kernel,
    out_type=(jax.ShapeDtypeStruct((B * L * S,), jnp.float32),
              jax.ShapeDtypeStruct((B * L * S,), jnp.float32)),
    mesh=plsc.VectorSubcoreMesh(core_axis_name="c", subcore_axis_name="s"),
    scratch_types=(
        pltpu.VMEM((CHUNKS, 128), jnp.int32),
        pltpu.VMEM((CHUNKS, 128), jnp.float32),
        pltpu.VMEM((CHUNKS, 128), jnp.float32),
        pltpu.VMEM((CHUNKS, 128), jnp.float32),
        pltpu.VMEM((CHUNKS, 128), jnp.float32),
        pltpu.VMEM((CHUNKS, 128), jnp.int32),
        pltpu.VMEM((CHUNKS, 128), jnp.float32),
        pltpu.VMEM((B * L // NC,), jnp.float32),
        pltpu.VMEM((16384,), jnp.float32),
        pltpu.VMEM_SHARED((B * L // NC,), jnp.float32),
        pltpu.SemaphoreType.DMA,
        pltpu.SemaphoreType.DMA,
    ),
)
def _sc_stage(a_hbm, fi_hbm, tg_hbm, out_hbm, tag_hbm, *scratch):
    _sc_body(a_hbm, fi_hbm, tg_hbm, out_hbm, tag_hbm, *scratch)


def kernel(M, lengths, edge_ind, W):
    del lengths
    alpha = _alpha_dense(M, W)

    flat = (jnp.arange(B, dtype=jnp.int32)[:, None] * (L * S)
            + edge_ind[:, :, 0] * S + edge_ind[:, :, 1])
    fi = flat.reshape(NC, NS, CHUNKS, 128)
    tg = lax.bitcast_convert_type(
        jnp.arange(NE, dtype=jnp.int32), jnp.float32
    ).reshape(NC, NS, CHUNKS, 128)

    out, _ = _sc_stage(alpha.reshape(-1), fi, tg)
    return out.reshape(B, L, S)


# speedup vs baseline: 2.6858x; 2.6858x over previous
import functools

import jax
import jax.numpy as jnp
from jax import lax
from jax.experimental import pallas as pl
from jax.experimental.pallas import tpu as pltpu
from jax.experimental.pallas import tpu_sc as plsc

S = 2048
B = 4
D = 128
L = 2048
E = 16384
NE = B * E

NC = 2
NS = 16
NW = NC * NS
EPT = NE // NW
CHUNKS = EPT // 128

RPC = B * L // NC
COLS = RPC // NS

LBLK = 512

_EPS = 1e-10

_SC_MESH = plsc.VectorSubcoreMesh(core_axis_name="c", subcore_axis_name="s")


def _tc_body(m_ref, w_ref, a_ref):
    b = pl.program_id(0)
    mb = m_ref[:, b, :]
    wb = w_ref[...]
    scale = lax.dot_general(wb, mb, (((1,), (1,)), ((), ())),
                            preferred_element_type=jnp.float32)
    mx = jnp.max(scale, axis=1, keepdims=True)
    e = jnp.exp(scale - mx)
    z = jnp.sum(e, axis=1, keepdims=True)
    a_ref[0] = e / z


def _alpha_dense(M, W):
    return pl.pallas_call(
        _tc_body,
        grid=(B, L // LBLK),
        in_specs=[
            pl.BlockSpec((S, B, D), lambda b, l: (0, 0, 0)),
            pl.BlockSpec((LBLK, D), lambda b, l: (l, 0)),
        ],
        out_specs=pl.BlockSpec((1, LBLK, S), lambda b, l: (b, l, 0)),
        out_shape=jax.ShapeDtypeStruct((B, L, S), jnp.float32),
    )(M, W)


def _sca_body(fi_hbm, tg_hbm, tag_hbm, fi_v, tg_v, sem):
    c = lax.axis_index("c")
    s = lax.axis_index("s")
    pltpu.sync_copy(fi_hbm.at[c, s], fi_v)
    pltpu.sync_copy(tg_hbm.at[c, s], tg_v)
    scats = [pltpu.async_copy(tg_v.at[k], tag_hbm.at[fi_v.at[k]], sem)
             for k in range(CHUNKS)]
    for g in scats:
        g.wait()


_sc_tags = pl.kernel(
    _sca_body,
    out_type=jax.ShapeDtypeStruct((B * L * S,), jnp.int32),
    mesh=_SC_MESH,
    compiler_params=pltpu.CompilerParams(needs_layout_passes=False),
    scratch_types=(
        pltpu.VMEM((CHUNKS, 128), jnp.int32),
        pltpu.VMEM((CHUNKS, 128), jnp.int32),
        pltpu.SemaphoreType.DMA,
    ),
)


def _scb_body(a_hbm, fi_hbm, tg_hbm, tag_hbm, out_hbm,
              fi_v, tg_v, al_v, wn_v, sc_v, dp_v, blk_v, df_v, dv_v, zb_v,
              stage_sh, dfin_sh, sem, zsem):
    c = lax.axis_index("c")
    s = lax.axis_index("s")

    def _zz(ref):
        def _z(i, carry):
            ref[pl.ds(i * 16, 16)] = jnp.zeros((16,), jnp.float32)
            return carry
        lax.fori_loop(0, ref.shape[0] // 16, _z, 0)
    _zz(zb_v)
    _zz(dp_v)

    pltpu.sync_copy(fi_hbm.at[c, s], fi_v)
    pltpu.sync_copy(tg_hbm.at[c, s], tg_v)

    gathers = [pltpu.async_copy(a_hbm.at[fi_v.at[k]], al_v.at[k], sem)
               for k in range(CHUNKS)]
    wins = [pltpu.async_copy(tag_hbm.at[fi_v.at[k]], wn_v.at[k], sem)
            for k in range(CHUNKS)]
    for g in gathers + wins:
        g.wait()

    row_base = c * RPC
    for k in range(CHUNKS):
        for j in range(8):
            sl = pl.ds(j * 16, 16)
            rl16 = lax.shift_right_logical(fi_v[k, sl], 11) - row_base
            canon = wn_v[k, sl] == tg_v[k, sl]
            ma16 = jnp.where(canon, al_v[k, sl], 0.0)
            plsc.addupdate_scatter(dp_v, [rl16], ma16)

    pltpu.sync_copy(dp_v, stage_sh.at[s])
    plsc.subcore_barrier()
    for r in range(NS):
        pltpu.sync_copy(stage_sh.at[r, pl.ds(s * COLS, COLS)], blk_v.at[r])

    def _red(i, carry):
        sl = pl.ds(i * 16, 16)
        acc = blk_v[0, sl]
        for r in range(1, NS):
            acc = acc + blk_v[r, sl]
        df_v[sl] = acc
        return carry
    lax.fori_loop(0, COLS // 16, _red, 0)
    pltpu.sync_copy(df_v, dfin_sh.at[pl.ds(s * COLS, COLS)])
    plsc.subcore_barrier()
    pltpu.sync_copy(dfin_sh, dv_v)

    for k in range(CHUNKS):
        for j in range(8):
            sl = pl.ds(j * 16, 16)
            rl16 = lax.shift_right_logical(fi_v[k, sl], 11) - row_base
            dval = plsc.load_gather(dv_v, [rl16])
            denom = dval * (1.0 - _EPS) + _EPS
            sc_v[k, sl] = al_v[k, sl] / denom

    zn = zb_v.shape[0]
    base = c * (B * L * S // NC) + s * (B * L * S // NW)
    nz = (B * L * S // NW) // zn
    zcopies = [pltpu.async_copy(zb_v, out_hbm.at[pl.ds(base + q * zn, zn)], zsem)
               for q in range(nz)]
    for g in zcopies:
        g.wait()
    plsc.subcore_barrier()

    outs = [pltpu.async_copy(sc_v.at[k], out_hbm.at[fi_v.at[k]], sem)
            for k in range(CHUNKS)]
    for g in outs:
        g.wait()


_sc_scores = pl.kernel(
    _scb_body,
    out_type=jax.ShapeDtypeStruct((B * L * S,), jnp.float32),
    mesh=_SC_MESH,
    compiler_params=pltpu.CompilerParams(needs_layout_passes=False),
    scratch_types=(
        pltpu.VMEM((CHUNKS, 128), jnp.int32),
        pltpu.VMEM((CHUNKS, 128), jnp.int32),
        pltpu.VMEM((CHUNKS, 128), jnp.float32),
        pltpu.VMEM((CHUNKS, 128), jnp.int32),
        pltpu.VMEM((CHUNKS, 128), jnp.float32),
        pltpu.VMEM((RPC,), jnp.float32),
        pltpu.VMEM((NS, COLS), jnp.float32),
        pltpu.VMEM((COLS,), jnp.float32),
        pltpu.VMEM((RPC,), jnp.float32),
        pltpu.VMEM((16384,), jnp.float32),
        pltpu.VMEM_SHARED((NS, RPC), jnp.float32),
        pltpu.VMEM_SHARED((RPC,), jnp.float32),
        pltpu.SemaphoreType.DMA,
        pltpu.SemaphoreType.DMA,
    ),
)


def _edge_stage(alpha_flat, fi, tg):
    tagbuf = _sc_tags(fi, tg)
    return _sc_scores(alpha_flat, fi, tg, tagbuf)


def kernel(M, lengths, edge_ind, W):
    del lengths
    alpha = _alpha_dense(M, W)

    flat = (jnp.arange(B, dtype=jnp.int32)[:, None] * (L * S)
            + edge_ind[:, :, 0] * S + edge_ind[:, :, 1])
    fi = flat.reshape(NC, NS, CHUNKS, 128)
    tg = jnp.arange(NE, dtype=jnp.int32).reshape(NC, NS, CHUNKS, 128)

    out = _edge_stage(alpha.reshape(-1), fi, tg)
    return out.reshape(B, L, S)


# speedup vs baseline: 4.5151x; 1.6811x over previous
import functools

import jax
import jax.numpy as jnp
from jax import lax
from jax.experimental import pallas as pl
from jax.experimental.pallas import tpu as pltpu
from jax.experimental.pallas import tpu_sc as plsc

S = 2048
B = 4
D = 128
L = 2048
E = 16384
NE = B * E

NC = 2
NS = 16
NW = NC * NS
EPT = NE // NW
CHUNKS = EPT // 128

RPC = B * L // NC
COLS = RPC // NS

LBLK = 512

_EPS = 1e-10

_SC_MESH = plsc.VectorSubcoreMesh(core_axis_name="c", subcore_axis_name="s")


def _tc_body(m_ref, w_ref, a_ref):
    b = pl.program_id(0)
    mb = m_ref[:, b, :]
    wb = w_ref[...]
    scale = lax.dot_general(wb, mb, (((1,), (1,)), ((), ())),
                            preferred_element_type=jnp.float32)
    mx = jnp.max(scale, axis=1, keepdims=True)
    e = jnp.exp(scale - mx)
    z = jnp.sum(e, axis=1, keepdims=True)
    a_ref[...] = (e / z).reshape(LBLK, S // 128, 128)


def _alpha_dense(M, W):
    return pl.pallas_call(
        _tc_body,
        grid=(B, L // LBLK),
        in_specs=[
            pl.BlockSpec((S, B, D), lambda b, l: (0, 0, 0)),
            pl.BlockSpec((LBLK, D), lambda b, l: (l, 0)),
        ],
        out_specs=pl.BlockSpec((LBLK, S // 128, 128),
                               lambda b, l: (b * (L // LBLK) + l, 0, 0)),
        out_shape=jax.ShapeDtypeStruct((B * L, S // 128, 128), jnp.float32),
    )(M, W)


def _sca_body(fi_hbm, tg_hbm, tag_hbm, fi_v, tg_v, sem):
    c = lax.axis_index("c")
    s = lax.axis_index("s")
    pltpu.sync_copy(fi_hbm.at[c, s], fi_v)
    pltpu.sync_copy(tg_hbm.at[c, s], tg_v)
    scats = [pltpu.async_copy(tg_v.at[k], tag_hbm.at[fi_v.at[k]], sem)
             for k in range(CHUNKS)]
    for g in scats:
        g.wait()


_sc_tags = pl.kernel(
    _sca_body,
    out_type=jax.ShapeDtypeStruct((B * L * S,), jnp.int32),
    mesh=_SC_MESH,
    compiler_params=pltpu.CompilerParams(needs_layout_passes=False),
    scratch_types=(
        pltpu.VMEM((CHUNKS, 128), jnp.int32),
        pltpu.VMEM((CHUNKS, 128), jnp.int32),
        pltpu.SemaphoreType.DMA,
    ),
)


def _scb_body(a_hbm, fi_hbm, tg_hbm, fo_hbm, tag_hbm, out_hbm,
              fi_v, tg_v, fo_v, al_v, wn_v, sc_v, dp_v, blk_v, df_v, dv_v,
              zb_v, stage_sh, dfin_sh, sem, zsem):
    c = lax.axis_index("c")
    s = lax.axis_index("s")

    def _zz(ref):
        def _z(i, carry):
            ref[pl.ds(i * 16, 16)] = jnp.zeros((16,), jnp.float32)
            return carry
        lax.fori_loop(0, ref.shape[0] // 16, _z, 0)
    _zz(zb_v)

    zn = zb_v.shape[0]
    zbase = c * (B * L * S // NC) + s * (B * L * S // NW)
    nz = (B * L * S // NW) // zn
    zcopies = [pltpu.async_copy(zb_v, out_hbm.at[pl.ds(zbase + q * zn, zn)],
                                zsem) for q in range(nz)]

    _zz(dp_v)

    pltpu.sync_copy(fi_hbm.at[c, s], fi_v)
    pltpu.sync_copy(tg_hbm.at[c, s], tg_v)
    pltpu.sync_copy(fo_hbm.at[c, s], fo_v)

    gathers = [pltpu.async_copy(a_hbm.at[fi_v.at[k]], al_v.at[k], sem)
               for k in range(CHUNKS)]
    wins = [pltpu.async_copy(tag_hbm.at[fi_v.at[k]], wn_v.at[k], sem)
            for k in range(CHUNKS)]
    for g in gathers + wins:
        g.wait()

    row_base = c * RPC
    for k in range(CHUNKS):
        for j in range(8):
            sl = pl.ds(j * 16, 16)
            rl16 = lax.shift_right_logical(fi_v[k, sl], 11) - row_base
            canon = wn_v[k, sl] == tg_v[k, sl]
            ma16 = jnp.where(canon, al_v[k, sl], 0.0)
            plsc.addupdate_scatter(dp_v, [rl16], ma16)

    pltpu.sync_copy(dp_v, stage_sh.at[s])
    plsc.subcore_barrier()
    for r in range(NS):
        pltpu.sync_copy(stage_sh.at[r, pl.ds(s * COLS, COLS)], blk_v.at[r])

    def _red(i, carry):
        sl = pl.ds(i * 16, 16)
        acc = blk_v[0, sl]
        for r in range(1, NS):
            acc = acc + blk_v[r, sl]
        df_v[sl] = acc
        return carry
    lax.fori_loop(0, COLS // 16, _red, 0)
    pltpu.sync_copy(df_v, dfin_sh.at[pl.ds(s * COLS, COLS)])
    plsc.subcore_barrier()
    pltpu.sync_copy(dfin_sh, dv_v)

    for k in range(CHUNKS):
        for j in range(8):
            sl = pl.ds(j * 16, 16)
            rl16 = lax.shift_right_logical(fi_v[k, sl], 11) - row_base
            dval = plsc.load_gather(dv_v, [rl16])
            denom = dval * (1.0 - _EPS) + _EPS
            sc_v[k, sl] = al_v[k, sl] / denom

    for g in zcopies:
        g.wait()
    plsc.subcore_barrier()

    outs = [pltpu.async_copy(sc_v.at[k], out_hbm.at[fo_v.at[k]], sem)
            for k in range(CHUNKS)]
    for g in outs:
        g.wait()


_sc_scores = pl.kernel(
    _scb_body,
    out_type=jax.ShapeDtypeStruct((B * L * S,), jnp.float32),
    mesh=_SC_MESH,
    compiler_params=pltpu.CompilerParams(needs_layout_passes=False),
    scratch_types=(
        pltpu.VMEM((CHUNKS, 128), jnp.int32),
        pltpu.VMEM((CHUNKS, 128), jnp.int32),
        pltpu.VMEM((CHUNKS, 128), jnp.int32),
        pltpu.VMEM((CHUNKS, 128), jnp.float32),
        pltpu.VMEM((CHUNKS, 128), jnp.int32),
        pltpu.VMEM((CHUNKS, 128), jnp.float32),
        pltpu.VMEM((RPC,), jnp.float32),
        pltpu.VMEM((NS, COLS), jnp.float32),
        pltpu.VMEM((COLS,), jnp.float32),
        pltpu.VMEM((RPC,), jnp.float32),
        pltpu.VMEM((16384,), jnp.float32),
        pltpu.VMEM_SHARED((NS, RPC), jnp.float32),
        pltpu.VMEM_SHARED((RPC,), jnp.float32),
        pltpu.SemaphoreType.DMA,
        pltpu.SemaphoreType.DMA,
    ),
)


def _edge_stage(alpha_flat, fi, tg, fo):
    tagbuf = _sc_tags(fi, tg)
    return _sc_scores(alpha_flat, fi, tg, fo, tagbuf)


def kernel(M, lengths, edge_ind, W):
    del lengths
    alpha = _alpha_dense(M, W)

    ei = edge_ind[:, :, 0]
    ej = edge_ind[:, :, 1]
    boff = jnp.arange(B, dtype=jnp.int32)[:, None]
    flat = boff * (L * S) + ei * S + ej
    tiled = (((boff * (L // 8) + (ei >> 3)) * (S // 128) + (ej >> 7)) * 1024
             + (ei & 7) * 128 + (ej & 127))
    fi = flat.reshape(NC, NS, CHUNKS, 128)
    fo = tiled.reshape(NC, NS, CHUNKS, 128)
    tg = jnp.arange(NE, dtype=jnp.int32).reshape(NC, NS, CHUNKS, 128)

    out = _edge_stage(alpha.reshape(-1), fi, tg, fo)
    return (out.reshape(B, L // 8, S // 128, 8, 128)
            .transpose(0, 1, 3, 2, 4)
            .reshape(B, L, S))


# speedup vs baseline: 4.5685x; 1.0118x over previous
import functools

import jax
import jax.numpy as jnp
from jax import lax
from jax.experimental import pallas as pl
from jax.experimental.pallas import tpu as pltpu
from jax.experimental.pallas import tpu_sc as plsc

S = 2048
B = 4
D = 128
L = 2048
E = 16384
NE = B * E

NC = 2
NS = 16
NW = NC * NS
EPT = NE // NW
CW = 128
CHUNKS = EPT // CW

RPC = B * L // NC
COLS = RPC // NS

LBLK = 512

_EPS = 1e-10

_SC_MESH = plsc.VectorSubcoreMesh(core_axis_name="c", subcore_axis_name="s")


def _tc_body(m_ref, w_ref, a_ref):
    b = pl.program_id(0)
    mb = m_ref[:, b, :]
    wb = w_ref[...]
    scale = lax.dot_general(wb, mb, (((1,), (1,)), ((), ())),
                            preferred_element_type=jnp.float32)
    mx = jnp.max(scale, axis=1, keepdims=True)
    e = jnp.exp(scale - mx)
    z = jnp.sum(e, axis=1, keepdims=True)
    a_ref[...] = (e / z).reshape(LBLK, S // 128, 128)


def _alpha_dense(M, W):
    return pl.pallas_call(
        _tc_body,
        grid=(B, L // LBLK),
        in_specs=[
            pl.BlockSpec((S, B, D), lambda b, l: (0, 0, 0)),
            pl.BlockSpec((LBLK, D), lambda b, l: (l, 0)),
        ],
        out_specs=pl.BlockSpec((LBLK, S // 128, 128),
                               lambda b, l: (b * (L // LBLK) + l, 0, 0)),
        out_shape=jax.ShapeDtypeStruct((B * L, S // 128, 128), jnp.float32),
    )(M, W)


def _sca_body(fi_hbm, tg_hbm, tag_hbm, fi_v, tg_v, sem):
    c = lax.axis_index("c")
    s = lax.axis_index("s")
    pltpu.sync_copy(fi_hbm.at[c, s], fi_v)
    pltpu.sync_copy(tg_hbm.at[c, s], tg_v)
    scats = [pltpu.async_copy(tg_v.at[k], tag_hbm.at[fi_v.at[k]], sem)
             for k in range(CHUNKS)]
    for g in scats:
        g.wait()


_sc_tags = pl.kernel(
    _sca_body,
    out_type=jax.ShapeDtypeStruct((B * L * S,), jnp.int32),
    mesh=_SC_MESH,
    compiler_params=pltpu.CompilerParams(needs_layout_passes=False),
    scratch_types=(
        pltpu.VMEM((CHUNKS, CW), jnp.int32),
        pltpu.VMEM((CHUNKS, CW), jnp.int32),
        pltpu.SemaphoreType.DMA,
    ),
)


def _scb_body(a_hbm, fi_hbm, tg_hbm, fo_hbm, tag_hbm, out_hbm,
              fi_v, tg_v, fo_v, al_v, wn_v, sc_v, dp_v, blk_v, df_v, dv_v,
              zb_v, stage_sh, dfin_sh, sem, zsem):
    c = lax.axis_index("c")
    s = lax.axis_index("s")

    def _zz(ref):
        def _z(i, carry):
            ref[pl.ds(i * 16, 16)] = jnp.zeros((16,), jnp.float32)
            return carry
        lax.fori_loop(0, ref.shape[0] // 16, _z, 0)
    _zz(zb_v)

    zn = zb_v.shape[0]
    zbase = c * (B * L * S // NC) + s * (B * L * S // NW)
    nz = (B * L * S // NW) // zn
    zcopies = [pltpu.async_copy(zb_v, out_hbm.at[pl.ds(zbase + q * zn, zn)],
                                zsem) for q in range(nz)]

    _zz(dp_v)

    pltpu.sync_copy(fi_hbm.at[c, s], fi_v)
    pltpu.sync_copy(tg_hbm.at[c, s], tg_v)
    pltpu.sync_copy(fo_hbm.at[c, s], fo_v)

    gathers = [pltpu.async_copy(a_hbm.at[fi_v.at[k]], al_v.at[k], sem)
               for k in range(CHUNKS)]
    wins = [pltpu.async_copy(tag_hbm.at[fi_v.at[k]], wn_v.at[k], sem)
            for k in range(CHUNKS)]
    for g in gathers + wins:
        g.wait()

    row_base = c * RPC
    for k in range(CHUNKS):
        for j in range(CW // 16):
            sl = pl.ds(j * 16, 16)
            rl16 = lax.shift_right_logical(fi_v[k, sl], 11) - row_base
            canon = wn_v[k, sl] == tg_v[k, sl]
            ma16 = jnp.where(canon, al_v[k, sl], 0.0)
            plsc.addupdate_scatter(dp_v, [rl16], ma16)

    pltpu.sync_copy(dp_v, stage_sh.at[s])
    plsc.subcore_barrier()
    blks = [pltpu.async_copy(stage_sh.at[r, pl.ds(s * COLS, COLS)],
                             blk_v.at[r], sem) for r in range(NS)]
    for g in blks:
        g.wait()

    def _red(i, carry):
        sl = pl.ds(i * 16, 16)
        acc = blk_v[0, sl]
        for r in range(1, NS):
            acc = acc + blk_v[r, sl]
        df_v[sl] = acc
        return carry
    lax.fori_loop(0, COLS // 16, _red, 0)
    pltpu.sync_copy(df_v, dfin_sh.at[pl.ds(s * COLS, COLS)])

    for g in zcopies:
        g.wait()
    plsc.subcore_barrier()
    pltpu.sync_copy(dfin_sh, dv_v)

    outs = []
    for k in range(CHUNKS):
        for j in range(CW // 16):
            sl = pl.ds(j * 16, 16)
            rl16 = lax.shift_right_logical(fi_v[k, sl], 11) - row_base
            dval = plsc.load_gather(dv_v, [rl16])
            denom = dval * (1.0 - _EPS) + _EPS
            sc_v[k, sl] = al_v[k, sl] / denom
        outs.append(pltpu.async_copy(sc_v.at[k], out_hbm.at[fo_v.at[k]], sem))
    for g in outs:
        g.wait()


_sc_scores = pl.kernel(
    _scb_body,
    out_type=jax.ShapeDtypeStruct((B * L * S,), jnp.float32),
    mesh=_SC_MESH,
    compiler_params=pltpu.CompilerParams(needs_layout_passes=False),
    scratch_types=(
        pltpu.VMEM((CHUNKS, CW), jnp.int32),
        pltpu.VMEM((CHUNKS, CW), jnp.int32),
        pltpu.VMEM((CHUNKS, CW), jnp.int32),
        pltpu.VMEM((CHUNKS, CW), jnp.float32),
        pltpu.VMEM((CHUNKS, CW), jnp.int32),
        pltpu.VMEM((CHUNKS, CW), jnp.float32),
        pltpu.VMEM((RPC,), jnp.float32),
        pltpu.VMEM((NS, COLS), jnp.float32),
        pltpu.VMEM((COLS,), jnp.float32),
        pltpu.VMEM((RPC,), jnp.float32),
        pltpu.VMEM((16384,), jnp.float32),
        pltpu.VMEM_SHARED((NS, RPC), jnp.float32),
        pltpu.VMEM_SHARED((RPC,), jnp.float32),
        pltpu.SemaphoreType.DMA,
        pltpu.SemaphoreType.DMA,
    ),
)


def _edge_stage(alpha_flat, fi, tg, fo):
    tagbuf = _sc_tags(fi, tg)
    return _sc_scores(alpha_flat, fi, tg, fo, tagbuf)


def kernel(M, lengths, edge_ind, W):
    del lengths
    alpha = _alpha_dense(M, W)

    ei = edge_ind[:, :, 0]
    ej = edge_ind[:, :, 1]
    boff = jnp.arange(B, dtype=jnp.int32)[:, None]
    flat = boff * (L * S) + ei * S + ej
    tiled = (((boff * (L // 8) + (ei >> 3)) * (S // 128) + (ej >> 7)) * 1024
             + (ei & 7) * 128 + (ej & 127))
    fi = flat.reshape(NC, NS, CHUNKS, CW)
    fo = tiled.reshape(NC, NS, CHUNKS, CW)
    tg = jnp.arange(NE, dtype=jnp.int32).reshape(NC, NS, CHUNKS, CW)

    out = _edge_stage(alpha.reshape(-1), fi, tg, fo)
    return (out.reshape(B, L // 8, S // 128, 8, 128)
            .transpose(0, 1, 3, 2, 4)
            .reshape(B, L, S))
